# baseline (speedup 1.0000x reference)
import jax
import jax.numpy as jnp
from jax import lax
from jax.experimental import pallas as pl
from jax.experimental.pallas import tpu as pltpu

TB1 = 16
TB2 = 64


def _conv1_kernel(xb_ref, w_ref, b_ref, o_ref):
    tb = xb_ref.shape[0]
    m = tb * 224
    x = xb_ref[...].reshape(m, 144)
    y0 = jnp.dot(x, w_ref[0], preferred_element_type=jnp.float32)
    y1 = jnp.dot(x, w_ref[1], preferred_element_type=jnp.float32)
    u0 = y0[:m - 1, :128] + y0[1:, 128:]
    u1 = y1[:m - 1, :128] + y1[1:, 128:]
    y = u0[:m - 17] + u1[16:]
    y = jnp.maximum(y + b_ref[...], 0.0)
    p = jnp.maximum(jnp.maximum(y[:, :32], y[:, 32:64]),
                    jnp.maximum(y[:, 64:96], y[:, 96:128])).astype(jnp.bfloat16)
    o_ref[: m - 17, :] = p
    o_ref[m - 17:, :] = jnp.zeros((17, 32), jnp.bfloat16)


def _head_kernel(p2_ref, w2_ref, b2_ref, w1_ref, b1_ref, wf_ref, bf_ref, o_ref):
    tb = p2_ref.shape[1]
    x = p2_ref[...].reshape(9 * tb, 800)
    y = jnp.dot(x, w2_ref[...], preferred_element_type=jnp.float32)
    y = jnp.maximum(y + b2_ref[...], 0.0)
    a = jnp.maximum(jnp.maximum(y[:, :64], y[:, 64:128]),
                    jnp.maximum(y[:, 128:192], y[:, 192:256]))
    a = a.reshape(9, tb, 64).astype(jnp.bfloat16)
    h = jnp.zeros((tb, 128), jnp.float32)
    for q in range(9):
        h = h + jnp.dot(a[q], w1_ref[q], preferred_element_type=jnp.float32)
    h = jnp.maximum(h + b1_ref[...], 0.0)
    o_ref[...] = jnp.dot(h, wf_ref[...],
                         preferred_element_type=jnp.float32) + bf_ref[...]


def _prep(conv1_w, conv1_b, conv2_w, conv2_b, fc1_w, fc1_b, fc2_w, fc2_b):
    w1_t = conv1_w.transpose(2, 3, 1, 0)
    cols = []
    for dy in (0, 1):
        for dx in (0, 1):
            m = jnp.pad(w1_t, ((3 * dy, 7 - 3 * dy), (3 * dx, 7 - 3 * dx),
                               (0, 0), (0, 0)))
            m = m.reshape(2, 6, 2, 6, 4, 32).transpose(0, 2, 4, 1, 3, 5)
            cols.append(m.reshape(576, 32))
    w1p = jnp.concatenate(cols, axis=1)
    w1s = w1p.reshape(2, 2, 144, 128)
    w1s = jnp.concatenate([w1s[:, 0], w1s[:, 1]], axis=2)
    b1p = jnp.tile(conv1_b, 4).reshape(1, 128)

    w2_t = conv2_w.transpose(2, 3, 1, 0)
    cols = []
    for dy in (0, 1):
        for dx in (0, 1):
            m = jnp.pad(w2_t, ((2 * dy, 2 - 2 * dy), (2 * dx, 2 - 2 * dx),
                               (0, 0), (0, 0)))
            cols.append(m.reshape(800, 64))
    w2p = jnp.concatenate(cols, axis=1)
    b2p = jnp.tile(conv2_b, 4).reshape(1, 256)

    w1r = fc1_w.T.reshape(64, 9, 128).transpose(1, 0, 2)
    bf1 = fc1_b.reshape(1, -1)
    wf2 = fc2_w.T
    bf2 = fc2_b.reshape(1, -1)
    return w1s, b1p, w2p, b2p, w1r, bf1, wf2, bf2


@jax.jit
def _forward(x_nchw, conv1_w, conv1_b, conv2_w, conv2_b,
             fc1_w, fc1_b, fc2_w, fc2_b):
    B = x_nchw.shape[0]
    A = fc2_w.shape[0]
    w1s, b1p, w2p, b2p, w1r, bf1, wf2, bf2 = _prep(
        conv1_w, conv1_b, conv2_w, conv2_b, fc1_w, fc1_b, fc2_w, fc2_b)

    tb = max(TB1, TB2)
    bp = -(-B // tb) * tb
    if bp != B:
        x_nchw = jnp.pad(x_nchw, ((0, bp - B), (0, 0), (0, 0), (0, 0)))

    xq = jnp.pad(x_nchw.astype(jnp.bfloat16),
                 ((0, 0), (0, 0), (0, 0), (0, 12)))
    xb = (xq.reshape(bp, 4, 14, 6, 16, 6)
            .transpose(0, 2, 4, 1, 3, 5)
            .reshape(bp, 14, 16, 144))

    a1 = pl.pallas_call(
        _conv1_kernel,
        out_shape=jax.ShapeDtypeStruct((bp * 224, 32), jnp.bfloat16),
        grid=(bp // TB1,),
        in_specs=[
            pl.BlockSpec((TB1, 14, 16, 144), lambda i: (i, 0, 0, 0)),
            pl.BlockSpec((2, 144, 256), lambda i: (0, 0, 0)),
            pl.BlockSpec((1, 128), lambda i: (0, 0)),
        ],
        out_specs=pl.BlockSpec((TB1 * 224, 32), lambda i: (i, 0)),
        compiler_params=pltpu.CompilerParams(
            dimension_semantics=("parallel",)),
    )(xb, w1s.astype(jnp.bfloat16), b1p)

    a1 = a1.reshape(bp, 14, 16, 32)
    p2 = jnp.stack(
        [a1[:, 4 * r:4 * r + 5, 4 * c:4 * c + 5, :].reshape(bp, 800)
         for r in range(3) for c in range(3)],
        axis=0)

    out = pl.pallas_call(
        _head_kernel,
        out_shape=jax.ShapeDtypeStruct((bp, A), jnp.float32),
        grid=(bp // TB2,),
        in_specs=[
            pl.BlockSpec((9, TB2, 800), lambda i: (0, i, 0)),
            pl.BlockSpec((800, 256), lambda i: (0, 0)),
            pl.BlockSpec((1, 256), lambda i: (0, 0)),
            pl.BlockSpec((9, 64, 128), lambda i: (0, 0, 0)),
            pl.BlockSpec((1, 128), lambda i: (0, 0)),
            pl.BlockSpec((128, A), lambda i: (0, 0)),
            pl.BlockSpec((1, A), lambda i: (0, 0)),
        ],
        out_specs=pl.BlockSpec((TB2, A), lambda i: (i, 0)),
        compiler_params=pltpu.CompilerParams(
            dimension_semantics=("parallel",)),
    )(p2, w2p.astype(jnp.bfloat16), b2p,
      w1r.astype(jnp.bfloat16), bf1, wf2, bf2)

    return out[:B]


def kernel(x_nchw, conv1_w, conv1_b, conv2_w, conv2_b,
           fc1_w, fc1_b, fc2_w, fc2_b):
    return _forward(x_nchw, conv1_w, conv1_b, conv2_w, conv2_b,
                    fc1_w, fc1_b, fc2_w, fc2_b)


# speedup vs baseline: 1.2732x; 1.2732x over previous
import jax
import jax.numpy as jnp
from jax.experimental import pallas as pl
from jax.experimental.pallas import tpu as pltpu

TB = 16


def _dqn_kernel(xb_ref, w_ref, b_ref, w2_ref, b2_ref, w1_ref, b1_ref,
                wf_ref, bf_ref, o_ref):
    tb = xb_ref.shape[0]
    m = tb * 224

    x = xb_ref[...].reshape(m, 144)
    y0 = jnp.dot(x, w_ref[0], preferred_element_type=jnp.float32)
    y1 = jnp.dot(x, w_ref[1], preferred_element_type=jnp.float32)
    u0 = y0[:m - 1, :128] + y0[1:, 128:]
    u1 = y1[:m - 1, :128] + y1[1:, 128:]
    y = u0[:m - 17] + u1[16:]
    y = jnp.maximum(y + b_ref[...], 0.0)
    p = jnp.maximum(jnp.maximum(y[:, :32], y[:, 32:64]),
                    jnp.maximum(y[:, 64:96], y[:, 96:128])).astype(jnp.bfloat16)
    a1 = jnp.concatenate([p, jnp.zeros((17, 32), jnp.bfloat16)], axis=0)
    a1 = a1.reshape(tb, 224, 32)

    x2 = jnp.concatenate(
        [jnp.concatenate(
            [a1[:, 64 * r + 16 * i + 4 * c + j, :]
             for i in range(5) for j in range(5)], axis=1)
         for r in range(3) for c in range(3)], axis=0)

    y2 = jnp.dot(x2, w2_ref[...], preferred_element_type=jnp.float32)
    y2 = jnp.maximum(y2 + b2_ref[...], 0.0)
    a2 = jnp.maximum(jnp.maximum(y2[:, :64], y2[:, 64:128]),
                     jnp.maximum(y2[:, 128:192], y2[:, 192:256]))
    a2 = a2.astype(jnp.bfloat16)

    af = jnp.concatenate([a2[q * tb:(q + 1) * tb, :] for q in range(9)],
                         axis=1)
    h = jnp.dot(af, w1_ref[...], preferred_element_type=jnp.float32)
    h = jnp.maximum(h + b1_ref[...], 0.0)
    o_ref[...] = jnp.dot(h, wf_ref[...],
                         preferred_element_type=jnp.float32) + bf_ref[...]


def _prep(conv1_w, conv1_b, conv2_w, conv2_b, fc1_w, fc1_b, fc2_w, fc2_b):
    w1_t = conv1_w.transpose(2, 3, 1, 0)
    cols = []
    for dy in (0, 1):
        for dx in (0, 1):
            m = jnp.pad(w1_t, ((3 * dy, 7 - 3 * dy), (3 * dx, 7 - 3 * dx),
                               (0, 0), (0, 0)))
            m = m.reshape(2, 6, 2, 6, 4, 32).transpose(0, 2, 4, 1, 3, 5)
            cols.append(m.reshape(576, 32))
    w1p = jnp.concatenate(cols, axis=1)
    w1s = w1p.reshape(2, 2, 144, 128)
    w1s = jnp.concatenate([w1s[:, 0], w1s[:, 1]], axis=2)
    b1p = jnp.tile(conv1_b, 4).reshape(1, 128)

    w2_t = conv2_w.transpose(2, 3, 1, 0)
    cols = []
    for dy in (0, 1):
        for dx in (0, 1):
            m = jnp.pad(w2_t, ((2 * dy, 2 - 2 * dy), (2 * dx, 2 - 2 * dx),
                               (0, 0), (0, 0)))
            cols.append(m.reshape(800, 64))
    w2p = jnp.concatenate(cols, axis=1)
    b2p = jnp.tile(conv2_b, 4).reshape(1, 256)

    w1r = fc1_w.T.reshape(64, 9, 128).transpose(1, 0, 2).reshape(576, 128)
    bf1 = fc1_b.reshape(1, -1)
    wf2 = fc2_w.T
    bf2 = fc2_b.reshape(1, -1)
    return w1s, b1p, w2p, b2p, w1r, bf1, wf2, bf2


@jax.jit
def _forward(x_nchw, conv1_w, conv1_b, conv2_w, conv2_b,
             fc1_w, fc1_b, fc2_w, fc2_b):
    B = x_nchw.shape[0]
    A = fc2_w.shape[0]
    w1s, b1p, w2p, b2p, w1r, bf1, wf2, bf2 = _prep(
        conv1_w, conv1_b, conv2_w, conv2_b, fc1_w, fc1_b, fc2_w, fc2_b)

    bp = -(-B // TB) * TB
    if bp != B:
        x_nchw = jnp.pad(x_nchw, ((0, bp - B), (0, 0), (0, 0), (0, 0)))

    xq = jnp.pad(x_nchw.astype(jnp.bfloat16),
                 ((0, 0), (0, 0), (0, 0), (0, 12)))
    xb = (xq.reshape(bp, 4, 14, 6, 16, 6)
            .transpose(0, 2, 4, 1, 3, 5)
            .reshape(bp, 14, 16, 144))

    out = pl.pallas_call(
        _dqn_kernel,
        out_shape=jax.ShapeDtypeStruct((bp, A), jnp.float32),
        grid=(bp // TB,),
        in_specs=[
            pl.BlockSpec((TB, 14, 16, 144), lambda i: (i, 0, 0, 0)),
            pl.BlockSpec((2, 144, 256), lambda i: (0, 0, 0)),
            pl.BlockSpec((1, 128), lambda i: (0, 0)),
            pl.BlockSpec((800, 256), lambda i: (0, 0)),
            pl.BlockSpec((1, 256), lambda i: (0, 0)),
            pl.BlockSpec((576, 128), lambda i: (0, 0)),
            pl.BlockSpec((1, 128), lambda i: (0, 0)),
            pl.BlockSpec((128, A), lambda i: (0, 0)),
            pl.BlockSpec((1, A), lambda i: (0, 0)),
        ],
        out_specs=pl.BlockSpec((TB, A), lambda i: (i, 0)),
        compiler_params=pltpu.CompilerParams(
            dimension_semantics=("parallel",)),
    )(xb, w1s.astype(jnp.bfloat16), b1p, w2p.astype(jnp.bfloat16), b2p,
      w1r.astype(jnp.bfloat16), bf1, wf2, bf2)

    return out[:B]


def kernel(x_nchw, conv1_w, conv1_b, conv2_w, conv2_b,
           fc1_w, fc1_b, fc2_w, fc2_b):
    return _forward(x_nchw, conv1_w, conv1_b, conv2_w, conv2_b,
                    fc1_w, fc1_b, fc2_w, fc2_b)


# speedup vs baseline: 1.3272x; 1.0424x over previous
import jax
import jax.numpy as jnp
from jax.experimental import pallas as pl
from jax.experimental.pallas import tpu as pltpu

TB = 32


def _dqn_kernel(xb_ref, w_ref, b_ref, w2_ref, b2_ref, w1_ref, b1_ref,
                wf_ref, bf_ref, o_ref):
    tb = xb_ref.shape[0]
    m = tb * 224

    x = xb_ref[...].reshape(m, 144)
    y0 = jnp.dot(x, w_ref[0], preferred_element_type=jnp.float32)
    y1 = jnp.dot(x, w_ref[1], preferred_element_type=jnp.float32)
    u0 = y0[:m - 1, :128] + y0[1:, 128:]
    u1 = y1[:m - 1, :128] + y1[1:, 128:]
    y = u0[:m - 17] + u1[16:]
    y = jnp.maximum(y + b_ref[...], 0.0)
    p = jnp.maximum(jnp.maximum(y[:, :32], y[:, 32:64]),
                    jnp.maximum(y[:, 64:96], y[:, 96:128])).astype(jnp.bfloat16)
    a1 = jnp.concatenate([p, jnp.zeros((17, 32), jnp.bfloat16)], axis=0)
    a1 = a1.reshape(tb, 224, 32)

    x2 = jnp.concatenate(
        [jnp.concatenate(
            [a1[:, 64 * r + 16 * i + 4 * c + j, :]
             for i in range(5) for j in range(5)], axis=1)
         for r in range(3) for c in range(3)], axis=0)

    y2 = jnp.dot(x2, w2_ref[...], preferred_element_type=jnp.float32)
    y2 = jnp.maximum(y2 + b2_ref[...], 0.0)
    a2 = jnp.maximum(jnp.maximum(y2[:, :64], y2[:, 64:128]),
                     jnp.maximum(y2[:, 128:192], y2[:, 192:256]))
    a2 = a2.astype(jnp.bfloat16)

    af = jnp.concatenate([a2[q * tb:(q + 1) * tb, :] for q in range(9)],
                         axis=1)
    h = jnp.dot(af, w1_ref[...], preferred_element_type=jnp.float32)
    h = jnp.maximum(h + b1_ref[...], 0.0)
    o_ref[...] = jnp.dot(h, wf_ref[...],
                         preferred_element_type=jnp.float32) + bf_ref[...]


def _prep(conv1_w, conv1_b, conv2_w, conv2_b, fc1_w, fc1_b, fc2_w, fc2_b):
    w1_t = conv1_w.transpose(2, 3, 1, 0)
    cols = []
    for dy in (0, 1):
        for dx in (0, 1):
            m = jnp.pad(w1_t, ((3 * dy, 7 - 3 * dy), (3 * dx, 7 - 3 * dx),
                               (0, 0), (0, 0)))
            m = m.reshape(2, 6, 2, 6, 4, 32).transpose(0, 2, 4, 1, 3, 5)
            cols.append(m.reshape(576, 32))
    w1p = jnp.concatenate(cols, axis=1)
    w1s = w1p.reshape(2, 2, 144, 128)
    w1s = jnp.concatenate([w1s[:, 0], w1s[:, 1]], axis=2)
    b1p = jnp.tile(conv1_b, 4).reshape(1, 128)

    w2_t = conv2_w.transpose(2, 3, 1, 0)
    cols = []
    for dy in (0, 1):
        for dx in (0, 1):
            m = jnp.pad(w2_t, ((2 * dy, 2 - 2 * dy), (2 * dx, 2 - 2 * dx),
                               (0, 0), (0, 0)))
            cols.append(m.reshape(800, 64))
    w2p = jnp.concatenate(cols, axis=1)
    b2p = jnp.tile(conv2_b, 4).reshape(1, 256)

    w1r = fc1_w.T.reshape(64, 9, 128).transpose(1, 0, 2).reshape(576, 128)
    bf1 = fc1_b.reshape(1, -1)
    wf2 = fc2_w.T
    bf2 = fc2_b.reshape(1, -1)
    return w1s, b1p, w2p, b2p, w1r, bf1, wf2, bf2


@jax.jit
def _forward(x_nchw, conv1_w, conv1_b, conv2_w, conv2_b,
             fc1_w, fc1_b, fc2_w, fc2_b):
    B = x_nchw.shape[0]
    A = fc2_w.shape[0]
    w1s, b1p, w2p, b2p, w1r, bf1, wf2, bf2 = _prep(
        conv1_w, conv1_b, conv2_w, conv2_b, fc1_w, fc1_b, fc2_w, fc2_b)

    bp = -(-B // TB) * TB
    if bp != B:
        x_nchw = jnp.pad(x_nchw, ((0, bp - B), (0, 0), (0, 0), (0, 0)))

    xq = jnp.pad(x_nchw.astype(jnp.bfloat16),
                 ((0, 0), (0, 0), (0, 0), (0, 12)))
    xb = (xq.reshape(bp, 4, 14, 6, 16, 6)
            .transpose(0, 2, 4, 1, 3, 5)
            .reshape(bp, 14, 16, 144))

    out = pl.pallas_call(
        _dqn_kernel,
        out_shape=jax.ShapeDtypeStruct((bp, A), jnp.float32),
        grid=(bp // TB,),
        in_specs=[
            pl.BlockSpec((TB, 14, 16, 144), lambda i: (i, 0, 0, 0)),
            pl.BlockSpec((2, 144, 256), lambda i: (0, 0, 0)),
            pl.BlockSpec((1, 128), lambda i: (0, 0)),
            pl.BlockSpec((800, 256), lambda i: (0, 0)),
            pl.BlockSpec((1, 256), lambda i: (0, 0)),
            pl.BlockSpec((576, 128), lambda i: (0, 0)),
            pl.BlockSpec((1, 128), lambda i: (0, 0)),
            pl.BlockSpec((128, A), lambda i: (0, 0)),
            pl.BlockSpec((1, A), lambda i: (0, 0)),
        ],
        out_specs=pl.BlockSpec((TB, A), lambda i: (i, 0)),
        compiler_params=pltpu.CompilerParams(
            dimension_semantics=("parallel",)),
    )(xb, w1s.astype(jnp.bfloat16), b1p, w2p.astype(jnp.bfloat16), b2p,
      w1r.astype(jnp.bfloat16), bf1, wf2, bf2)

    return out[:B]


def kernel(x_nchw, conv1_w, conv1_b, conv2_w, conv2_b,
           fc1_w, fc1_b, fc2_w, fc2_b):
    return _forward(x_nchw, conv1_w, conv1_b, conv2_w, conv2_b,
                    fc1_w, fc1_b, fc2_w, fc2_b)


# speedup vs baseline: 1.3379x; 1.0080x over previous
import jax
import jax.numpy as jnp
from jax.experimental import pallas as pl
from jax.experimental.pallas import tpu as pltpu

TB = 64


def _dqn_kernel(xb_ref, w_ref, b_ref, w2_ref, b2_ref, w1_ref, b1_ref,
                wf_ref, bf_ref, o_ref):
    tb = xb_ref.shape[0]
    m = tb * 224

    x = xb_ref[...].reshape(m, 144)
    y0 = jnp.dot(x, w_ref[0], preferred_element_type=jnp.float32)
    y1 = jnp.dot(x, w_ref[1], preferred_element_type=jnp.float32)
    u0 = y0[:m - 1, :128] + y0[1:, 128:]
    u1 = y1[:m - 1, :128] + y1[1:, 128:]
    y = u0[:m - 17] + u1[16:]
    y = jnp.maximum(y + b_ref[...], 0.0)
    p = jnp.maximum(jnp.maximum(y[:, :32], y[:, 32:64]),
                    jnp.maximum(y[:, 64:96], y[:, 96:128])).astype(jnp.bfloat16)
    a1 = jnp.concatenate([p, jnp.zeros((17, 32), jnp.bfloat16)], axis=0)
    a1 = a1.reshape(tb, 224, 32)

    x2 = jnp.concatenate(
        [jnp.concatenate(
            [a1[:, 64 * r + 16 * i + 4 * c + j, :]
             for i in range(5) for j in range(5)], axis=1)
         for r in range(3) for c in range(3)], axis=0)

    y2 = jnp.dot(x2, w2_ref[...], preferred_element_type=jnp.float32)
    y2 = jnp.maximum(y2 + b2_ref[...], 0.0)
    a2 = jnp.maximum(jnp.maximum(y2[:, :64], y2[:, 64:128]),
                     jnp.maximum(y2[:, 128:192], y2[:, 192:256]))
    a2 = a2.astype(jnp.bfloat16)

    af = jnp.concatenate([a2[q * tb:(q + 1) * tb, :] for q in range(9)],
                         axis=1)
    h = jnp.dot(af, w1_ref[...], preferred_element_type=jnp.float32)
    h = jnp.maximum(h + b1_ref[...], 0.0)
    o_ref[...] = jnp.dot(h, wf_ref[...],
                         preferred_element_type=jnp.float32) + bf_ref[...]


def _prep(conv1_w, conv1_b, conv2_w, conv2_b, fc1_w, fc1_b, fc2_w, fc2_b):
    w1_t = conv1_w.transpose(2, 3, 1, 0)
    cols = []
    for dy in (0, 1):
        for dx in (0, 1):
            m = jnp.pad(w1_t, ((3 * dy, 7 - 3 * dy), (3 * dx, 7 - 3 * dx),
                               (0, 0), (0, 0)))
            m = m.reshape(2, 6, 2, 6, 4, 32).transpose(0, 2, 4, 1, 3, 5)
            cols.append(m.reshape(576, 32))
    w1p = jnp.concatenate(cols, axis=1)
    w1s = w1p.reshape(2, 2, 144, 128)
    w1s = jnp.concatenate([w1s[:, 0], w1s[:, 1]], axis=2)
    b1p = jnp.tile(conv1_b, 4).reshape(1, 128)

    w2_t = conv2_w.transpose(2, 3, 1, 0)
    cols = []
    for dy in (0, 1):
        for dx in (0, 1):
            m = jnp.pad(w2_t, ((2 * dy, 2 - 2 * dy), (2 * dx, 2 - 2 * dx),
                               (0, 0), (0, 0)))
            cols.append(m.reshape(800, 64))
    w2p = jnp.concatenate(cols, axis=1)
    b2p = jnp.tile(conv2_b, 4).reshape(1, 256)

    w1r = fc1_w.T.reshape(64, 9, 128).transpose(1, 0, 2).reshape(576, 128)
    bf1 = fc1_b.reshape(1, -1)
    wf2 = fc2_w.T
    bf2 = fc2_b.reshape(1, -1)
    return w1s, b1p, w2p, b2p, w1r, bf1, wf2, bf2


@jax.jit
def _forward(x_nchw, conv1_w, conv1_b, conv2_w, conv2_b,
             fc1_w, fc1_b, fc2_w, fc2_b):
    B = x_nchw.shape[0]
    A = fc2_w.shape[0]
    w1s, b1p, w2p, b2p, w1r, bf1, wf2, bf2 = _prep(
        conv1_w, conv1_b, conv2_w, conv2_b, fc1_w, fc1_b, fc2_w, fc2_b)

    bp = -(-B // TB) * TB
    if bp != B:
        x_nchw = jnp.pad(x_nchw, ((0, bp - B), (0, 0), (0, 0), (0, 0)))

    xq = jnp.pad(x_nchw.astype(jnp.bfloat16),
                 ((0, 0), (0, 0), (0, 0), (0, 12)))
    xb = (xq.reshape(bp, 4, 14, 6, 16, 6)
            .transpose(0, 2, 4, 1, 3, 5)
            .reshape(bp, 14, 16, 144))

    out = pl.pallas_call(
        _dqn_kernel,
        out_shape=jax.ShapeDtypeStruct((bp, A), jnp.float32),
        grid=(bp // TB,),
        in_specs=[
            pl.BlockSpec((TB, 14, 16, 144), lambda i: (i, 0, 0, 0)),
            pl.BlockSpec((2, 144, 256), lambda i: (0, 0, 0)),
            pl.BlockSpec((1, 128), lambda i: (0, 0)),
            pl.BlockSpec((800, 256), lambda i: (0, 0)),
            pl.BlockSpec((1, 256), lambda i: (0, 0)),
            pl.BlockSpec((576, 128), lambda i: (0, 0)),
            pl.BlockSpec((1, 128), lambda i: (0, 0)),
            pl.BlockSpec((128, A), lambda i: (0, 0)),
            pl.BlockSpec((1, A), lambda i: (0, 0)),
        ],
        out_specs=pl.BlockSpec((TB, A), lambda i: (i, 0)),
        compiler_params=pltpu.CompilerParams(
            dimension_semantics=("parallel",)),
    )(xb, w1s.astype(jnp.bfloat16), b1p, w2p.astype(jnp.bfloat16), b2p,
      w1r.astype(jnp.bfloat16), bf1, wf2, bf2)

    return out[:B]


def kernel(x_nchw, conv1_w, conv1_b, conv2_w, conv2_b,
           fc1_w, fc1_b, fc2_w, fc2_b):
    return _forward(x_nchw, conv1_w, conv1_b, conv2_w, conv2_b,
                    fc1_w, fc1_b, fc2_w, fc2_b)
